# baseline (speedup 1.0000x reference)
import functools

import jax
import jax.numpy as jnp
from jax import lax
from jax.experimental import pallas as pl
from jax.experimental.pallas import tpu as pltpu
from jax.experimental.pallas import tpu_sc as plsc

NC = 2
NS = 16
NW = NC * NS
L = 16

D = 1024
K = 32


def _sc_body(x_hbm, idx_hbm, tab_hbm, out_hbm, idx_v, rows_v, xb_v):
    b_per_w = idx_v.shape[0]
    n_chunks = b_per_w // K
    wid = lax.axis_index("s") * NC + lax.axis_index("c")
    base = wid * b_per_w

    pltpu.sync_copy(idx_hbm.at[pl.ds(base, b_per_w)], idx_v)

    @pl.loop(0, n_chunks)
    def _chunk(c):
        row0 = base + c * K
        pltpu.sync_copy(tab_hbm.at[idx_v.at[pl.ds(c * K, K)]], rows_v)
        pltpu.sync_copy(x_hbm.at[pl.ds(row0 * D, K * D)], xb_v)

        @pl.loop(0, K)
        def _row(r):
            for i in range(D // L):
                o = r * D + i * L
                xb_v[pl.ds(o, L)] = xb_v[pl.ds(o, L)] + rows_v[r, pl.ds(i * L, L)]

        pltpu.sync_copy(xb_v, out_hbm.at[pl.ds(row0 * D, K * D)])


def _sc_call(xf, idx, pos_table):
    B = idx.shape[0]
    b_per_w = B // NW
    mesh = plsc.VectorSubcoreMesh(core_axis_name="c", subcore_axis_name="s")
    k = pl.kernel(
        _sc_body,
        out_type=jax.ShapeDtypeStruct((B * D,), jnp.float32),
        mesh=mesh,
        scratch_types=[
            pltpu.VMEM((b_per_w,), jnp.int32),
            pltpu.VMEM((K, D), jnp.float32),
            pltpu.VMEM((K * D,), jnp.float32),
        ],
    )
    return k(xf, idx, pos_table)


@jax.jit
def kernel(x, n, pos_table):
    b, s, d = x.shape
    xf = x.reshape(b * s * d)
    idx = n.reshape(b * s).astype(jnp.int32)
    out = _sc_call(xf, idx, pos_table)
    return out.reshape(b, s, d)


# speedup vs baseline: 1.4639x; 1.4639x over previous
import functools

import jax
import jax.numpy as jnp
from jax import lax
from jax.experimental import pallas as pl
from jax.experimental.pallas import tpu as pltpu
from jax.experimental.pallas import tpu_sc as plsc

NC = 2
NS = 16
NW = NC * NS
L = 16

D = 1024
K = 16


def _sc_body(x_hbm, idx_hbm, tab_hbm, out_hbm,
             idx_v, rows0, rows1, xb0, xb1,
             sg0, sg1, sx0, sx1, so0, so1):
    b_per_w = idx_v.shape[0]
    n_chunks = b_per_w // K
    wid = lax.axis_index("s") * NC + lax.axis_index("c")
    base = wid * b_per_w

    pltpu.sync_copy(idx_hbm.at[pl.ds(base, b_per_w)], idx_v)

    def issue_loads(c, rows_b, xb_b, sg, sx):
        row0 = base + c * K
        dx = pltpu.async_copy(x_hbm.at[pl.ds(row0 * D, K * D)], xb_b, sx)
        dg = pltpu.async_copy(tab_hbm.at[idx_v.at[pl.ds(c * K, K)]], rows_b, sg)
        return dx, dg

    def add_chunk(rows_b, xb_b):
        @plsc.parallel_loop(0, K)
        def _row(r):
            o = r * D
            for i in range(D // L):
                xb_b[pl.ds(o + i * L, L)] = (
                    xb_b[pl.ds(o + i * L, L)] + rows_b[r, pl.ds(i * L, L)]
                )

    def drain_store(xb_b, so):
        pltpu.make_async_copy(x_hbm.at[pl.ds(base * D, K * D)], xb_b, so).wait()

    @pl.loop(0, n_chunks, step=2)
    def _pair(g):
        @pl.when(g > 0)
        def _():
            drain_store(xb0, so0)
            drain_store(xb1, so1)

        dx0, dg0 = issue_loads(g, rows0, xb0, sg0, sx0)
        dx1, dg1 = issue_loads(g + 1, rows1, xb1, sg1, sx1)

        dx0.wait()
        dg0.wait()
        add_chunk(rows0, xb0)
        pltpu.async_copy(xb0, out_hbm.at[pl.ds((base + g * K) * D, K * D)], so0)

        dx1.wait()
        dg1.wait()
        add_chunk(rows1, xb1)
        pltpu.async_copy(xb1, out_hbm.at[pl.ds((base + (g + 1) * K) * D, K * D)], so1)

    drain_store(xb0, so0)
    drain_store(xb1, so1)


def _sc_call(xf, idx, pos_table):
    B = idx.shape[0]
    b_per_w = B // NW
    mesh = plsc.VectorSubcoreMesh(core_axis_name="c", subcore_axis_name="s")
    k = pl.kernel(
        _sc_body,
        out_type=jax.ShapeDtypeStruct((B * D,), jnp.float32),
        mesh=mesh,
        scratch_types=[
            pltpu.VMEM((b_per_w,), jnp.int32),
            pltpu.VMEM((K, D), jnp.float32),
            pltpu.VMEM((K, D), jnp.float32),
            pltpu.VMEM((K * D,), jnp.float32),
            pltpu.VMEM((K * D,), jnp.float32),
            pltpu.SemaphoreType.DMA,
            pltpu.SemaphoreType.DMA,
            pltpu.SemaphoreType.DMA,
            pltpu.SemaphoreType.DMA,
            pltpu.SemaphoreType.DMA,
            pltpu.SemaphoreType.DMA,
        ],
    )
    return k(xf, idx, pos_table)


@jax.jit
def kernel(x, n, pos_table):
    b, s, d = x.shape
    xf = x.reshape(b * s * d)
    idx = n.reshape(b * s).astype(jnp.int32)
    out = _sc_call(xf, idx, pos_table)
    return out.reshape(b, s, d)


# speedup vs baseline: 2.2367x; 1.5279x over previous
import jax
import jax.numpy as jnp
from jax import lax
from jax.experimental import pallas as pl
from jax.experimental.pallas import tpu as pltpu
from jax.experimental.pallas import tpu_sc as plsc

NC = 2
NS = 16
NW = NC * NS
L = 16

D = 1024
K = 16
CPR = D // 128
NCK = K * CPR


def _sc_body(x_hbm, gidx_hbm, tab_hbm, out_hbm,
             gidx_v, rows0, rows1, xb0, xb1,
             sg0, sg1, sx0, sx1, so0, so1):
    b_per_w = gidx_v.shape[0] // CPR
    n_chunks = b_per_w // K
    wid = lax.axis_index("s") * NC + lax.axis_index("c")
    base = wid * b_per_w

    pltpu.sync_copy(gidx_hbm.at[pl.ds(base * CPR, b_per_w * CPR)], gidx_v)

    def issue_loads(c, rows_b, xb_b, sg, sx):
        dx = pltpu.async_copy(x_hbm.at[pl.ds(base + c * K, K)], xb_b, sx)
        dg = pltpu.async_copy(tab_hbm.at[gidx_v.at[pl.ds(c * NCK, NCK)]],
                              rows_b, sg)
        return dx, dg

    def add_chunk(rows_b, xb_b):
        @plsc.parallel_loop(0, K)
        def _r(r):
            v0 = r * CPR
            for t in range(CPR):
                for c in range(128 // L):
                    xb_b[r, pl.ds(t * 128 + c * L, L)] = (
                        xb_b[r, pl.ds(t * 128 + c * L, L)]
                        + rows_b[v0 + t, pl.ds(c * L, L)]
                    )

    def drain_store(xb_b, so):
        pltpu.make_async_copy(x_hbm.at[pl.ds(base, K)], xb_b, so).wait()

    @pl.loop(0, n_chunks, step=2)
    def _pair(g):
        @pl.when(g > 0)
        def _():
            drain_store(xb0, so0)
            drain_store(xb1, so1)

        dx0, dg0 = issue_loads(g, rows0, xb0, sg0, sx0)
        dx1, dg1 = issue_loads(g + 1, rows1, xb1, sg1, sx1)

        dx0.wait()
        dg0.wait()
        add_chunk(rows0, xb0)
        pltpu.async_copy(xb0, out_hbm.at[pl.ds(base + g * K, K)], so0)

        dx1.wait()
        dg1.wait()
        add_chunk(rows1, xb1)
        pltpu.async_copy(xb1, out_hbm.at[pl.ds(base + (g + 1) * K, K)], so1)

    drain_store(xb0, so0)
    drain_store(xb1, so1)


def _sc_call(x2, gidx, tab2):
    B = x2.shape[0]
    b_per_w = B // NW
    mesh = plsc.VectorSubcoreMesh(core_axis_name="c", subcore_axis_name="s")
    k = pl.kernel(
        _sc_body,
        out_type=jax.ShapeDtypeStruct((B, D), jnp.float32),
        mesh=mesh,
        scratch_types=[
            pltpu.VMEM((b_per_w * CPR,), jnp.int32),
            pltpu.VMEM((NCK, 128), jnp.float32),
            pltpu.VMEM((NCK, 128), jnp.float32),
            pltpu.VMEM((K, D), jnp.float32),
            pltpu.VMEM((K, D), jnp.float32),
            pltpu.SemaphoreType.DMA,
            pltpu.SemaphoreType.DMA,
            pltpu.SemaphoreType.DMA,
            pltpu.SemaphoreType.DMA,
            pltpu.SemaphoreType.DMA,
            pltpu.SemaphoreType.DMA,
        ],
    )
    return k(x2, gidx, tab2)


@jax.jit
def kernel(x, n, pos_table):
    b, s, d = x.shape
    x2 = x.reshape(b * s, d)
    idx = n.reshape(b * s).astype(jnp.int32)
    gidx = (idx[:, None] * CPR + jnp.arange(CPR, dtype=jnp.int32)).reshape(-1)
    tab2 = pos_table.reshape(pos_table.shape[0] * CPR, 128)
    out = _sc_call(x2, gidx, tab2)
    return out.reshape(b, s, d)


# speedup vs baseline: 3.0804x; 1.3772x over previous
import jax
import jax.numpy as jnp
from jax import lax
from jax.experimental import pallas as pl
from jax.experimental.pallas import tpu as pltpu
from jax.experimental.pallas import tpu_sc as plsc

NC = 2
NS = 16
NW = NC * NS
L = 16

D = 1024
K = 16


def _sc_body(x_hbm, idx_hbm, tab_hbm, out_hbm,
             idx_v, rows0, rows1, xb0, xb1,
             sg0, sg1, sx0, sx1, so0, so1):
    b_per_w = idx_v.shape[0]
    n_chunks = b_per_w // K
    wid = lax.axis_index("s") * NC + lax.axis_index("c")
    base = wid * b_per_w

    pltpu.sync_copy(idx_hbm.at[pl.ds(base, b_per_w)], idx_v)

    def issue_loads(c, rows_b, xb_b, sg, sx):
        dx = pltpu.async_copy(x_hbm.at[pl.ds(base + c * K, K)], xb_b, sx)
        dg = pltpu.async_copy(tab_hbm.at[idx_v.at[pl.ds(c * K, K)]],
                              rows_b, sg)
        return dx, dg

    def add_chunk(rows_b, xb_b):
        @plsc.parallel_loop(0, K)
        def _r(r):
            for c in range(D // L):
                xb_b[r, pl.ds(c * L, L)] = (
                    xb_b[r, pl.ds(c * L, L)] + rows_b[r, pl.ds(c * L, L)]
                )

    def drain_store(xb_b, so):
        pltpu.make_async_copy(x_hbm.at[pl.ds(base, K)], xb_b, so).wait()

    @pl.loop(0, n_chunks, step=2)
    def _pair(g):
        @pl.when(g > 0)
        def _():
            drain_store(xb0, so0)
            drain_store(xb1, so1)

        dx0, dg0 = issue_loads(g, rows0, xb0, sg0, sx0)
        dx1, dg1 = issue_loads(g + 1, rows1, xb1, sg1, sx1)

        dx0.wait()
        dg0.wait()
        add_chunk(rows0, xb0)
        pltpu.async_copy(xb0, out_hbm.at[pl.ds(base + g * K, K)], so0)

        dx1.wait()
        dg1.wait()
        add_chunk(rows1, xb1)
        pltpu.async_copy(xb1, out_hbm.at[pl.ds(base + (g + 1) * K, K)], so1)

    drain_store(xb0, so0)
    drain_store(xb1, so1)


def _sc_call(x2, idx, tab):
    B = x2.shape[0]
    b_per_w = B // NW
    mesh = plsc.VectorSubcoreMesh(core_axis_name="c", subcore_axis_name="s")
    k = pl.kernel(
        _sc_body,
        out_type=jax.ShapeDtypeStruct((B, D), jnp.float32),
        mesh=mesh,
        scratch_types=[
            pltpu.VMEM((b_per_w,), jnp.int32),
            pltpu.VMEM((K, D), jnp.float32),
            pltpu.VMEM((K, D), jnp.float32),
            pltpu.VMEM((K, D), jnp.float32),
            pltpu.VMEM((K, D), jnp.float32),
            pltpu.SemaphoreType.DMA,
            pltpu.SemaphoreType.DMA,
            pltpu.SemaphoreType.DMA,
            pltpu.SemaphoreType.DMA,
            pltpu.SemaphoreType.DMA,
            pltpu.SemaphoreType.DMA,
        ],
    )
    return k(x2, idx, tab)


@jax.jit
def kernel(x, n, pos_table):
    b, s, d = x.shape
    x2 = x.reshape(b * s, d)
    idx = n.reshape(b * s).astype(jnp.int32)
    out = _sc_call(x2, idx, pos_table)
    return out.reshape(b, s, d)
